# baseline (speedup 1.0000x reference)
import functools

import jax
import jax.numpy as jnp
from jax import lax
from jax.experimental import pallas as pl
from jax.experimental.pallas import tpu as pltpu
from jax.experimental.pallas import tpu_sc as plsc

VOCAB = 100000
DIM = 128
B = 4096
L = 200

_info = plsc.get_sparse_core_info()
NC, NS = _info.num_cores, _info.num_subcores
NW = NC * NS

TOTAL = B * L
CHUNK = 128
SC_CHUNKS_PER_W = 184
PER_W = SC_CHUNKS_PER_W * CHUNK
S_SC = PER_W * NW
N_TAIL = TOTAL - S_SC
NBUF = 4
N_GROUPS = SC_CHUNKS_PER_W // NBUF


def _make_gather():
    mesh = plsc.VectorSubcoreMesh(core_axis_name="c", subcore_axis_name="s")

    @functools.partial(
        pl.kernel,
        mesh=mesh,
        out_type=jax.ShapeDtypeStruct((TOTAL, DIM), jnp.float32),
        scratch_types=[
            pltpu.VMEM((SC_CHUNKS_PER_W, CHUNK), jnp.int32),
            pltpu.VMEM((NBUF, CHUNK, DIM), jnp.float32),
            pltpu.SemaphoreType.DMA((NBUF,)),
            pltpu.SemaphoreType.DMA((NBUF,)),
        ],
    )
    def gather_kernel(table_hbm, ids_hbm, out_hbm, idx_all, rows, sem_g, sem_s):
        wid = lax.axis_index("s") * NC + lax.axis_index("c")
        base = wid * PER_W

        pltpu.sync_copy(ids_hbm.at[wid], idx_all)

        def gather(k, b):
            pltpu.async_copy(table_hbm.at[idx_all.at[k]], rows.at[b], sem_g.at[b])

        def wait_gather(k, b):
            pltpu.make_async_copy(
                table_hbm.at[idx_all.at[k]], rows.at[b], sem_g.at[b]).wait()

        def store(k, b):
            pltpu.async_copy(
                rows.at[b], out_hbm.at[pl.ds(base + k * CHUNK, CHUNK)], sem_s.at[b])

        def wait_store(k, b):
            pltpu.make_async_copy(
                rows.at[b], out_hbm.at[pl.ds(base + k * CHUNK, CHUNK)],
                sem_s.at[b]).wait()

        for b in range(NBUF):
            gather(b, b)
        for b in range(NBUF):
            wait_gather(b, b)
            store(b, b)

        def body(j, _):
            k0 = j * NBUF
            for b in range(NBUF):
                k = k0 + b
                wait_store(k - NBUF, b)
                gather(k, b)
            for b in range(NBUF):
                k = k0 + b
                wait_gather(k, b)
                store(k, b)
            return 0

        lax.fori_loop(1, N_GROUPS, body, 0)

        for b in range(NBUF):
            wait_store(SC_CHUNKS_PER_W - NBUF + b, b)

    return gather_kernel


_gather = _make_gather()


def kernel(input_ids, table):
    ids_flat = input_ids.reshape(TOTAL).astype(jnp.int32)
    ids_head = ids_flat[:S_SC].reshape(NW, SC_CHUNKS_PER_W, CHUNK)
    tail = jnp.take(table, ids_flat[S_SC:], axis=0)
    out_full = _gather(table, ids_head)
    out = lax.dynamic_update_slice(out_full, tail, (S_SC, 0))
    return out.reshape(B, L, DIM)


# speedup vs baseline: 1.0992x; 1.0992x over previous
import functools

import jax
import jax.numpy as jnp
from jax import lax
from jax.experimental import pallas as pl
from jax.experimental.pallas import tpu as pltpu
from jax.experimental.pallas import tpu_sc as plsc

VOCAB = 100000
DIM = 128
B = 4096
L = 200

_info = plsc.get_sparse_core_info()
NC, NS = _info.num_cores, _info.num_subcores
NW = NC * NS

TOTAL = B * L
PER_W = TOTAL // NW
CHUNK = 128
N_CHUNKS = PER_W // CHUNK
NBUF = 4
LAG = 2
N_GROUPS = N_CHUNKS // NBUF


def _make_gather():
    mesh = plsc.VectorSubcoreMesh(core_axis_name="c", subcore_axis_name="s")

    @functools.partial(
        pl.kernel,
        mesh=mesh,
        out_type=jax.ShapeDtypeStruct((TOTAL, DIM), jnp.float32),
        scratch_types=[
            pltpu.VMEM((N_CHUNKS, CHUNK), jnp.int32),
            pltpu.VMEM((NBUF, CHUNK, DIM), jnp.float32),
            pltpu.SemaphoreType.DMA((NBUF,)),
            pltpu.SemaphoreType.DMA((NBUF,)),
        ],
    )
    def gather_kernel(table_hbm, ids_hbm, out_hbm, idx_all, rows, sem_g, sem_s):
        wid = lax.axis_index("s") * NC + lax.axis_index("c")
        base = wid * PER_W

        pltpu.sync_copy(ids_hbm.at[wid], idx_all)

        def gather(k, b):
            pltpu.async_copy(table_hbm.at[idx_all.at[k]], rows.at[b], sem_g.at[b])

        def wait_gather(k, b):
            pltpu.make_async_copy(
                table_hbm.at[idx_all.at[k]], rows.at[b], sem_g.at[b]).wait()

        def store(k, b):
            pltpu.async_copy(
                rows.at[b], out_hbm.at[pl.ds(base + k * CHUNK, CHUNK)], sem_s.at[b])

        def wait_store(k, b):
            pltpu.make_async_copy(
                rows.at[b], out_hbm.at[pl.ds(base + k * CHUNK, CHUNK)],
                sem_s.at[b]).wait()

        for k in range(LAG):
            gather(k, k)
        for k in range(LAG, NBUF):
            gather(k, k)
            wait_gather(k - LAG, k - LAG)
            store(k - LAG, k - LAG)

        def body(j, _):
            k0 = j * NBUF
            for b in range(NBUF):
                k = k0 + b
                wait_store(k - NBUF, b)
                gather(k, b)
                kl = k - LAG
                bl = (b - LAG) % NBUF
                wait_gather(kl, bl)
                store(kl, bl)
            return 0

        lax.fori_loop(1, N_GROUPS, body, 0)

        for k in range(N_CHUNKS - LAG, N_CHUNKS):
            wait_gather(k, k % NBUF)
            store(k, k % NBUF)
        for k in range(N_CHUNKS - NBUF, N_CHUNKS):
            wait_store(k, k % NBUF)

    return gather_kernel


_gather = _make_gather()


def kernel(input_ids, table):
    ids = input_ids.reshape(NW, N_CHUNKS, CHUNK).astype(jnp.int32)
    out_flat = _gather(table, ids)
    return out_flat.reshape(B, L, DIM)


# speedup vs baseline: 1.1012x; 1.0018x over previous
import functools

import jax
import jax.numpy as jnp
from jax import lax
from jax.experimental import pallas as pl
from jax.experimental.pallas import tpu as pltpu
from jax.experimental.pallas import tpu_sc as plsc

VOCAB = 100000
DIM = 128
B = 4096
L = 200

_info = plsc.get_sparse_core_info()
NC, NS = _info.num_cores, _info.num_subcores
NW = NC * NS

TOTAL = B * L
PER_W = TOTAL // NW
CHUNK = 128
N_CHUNKS = PER_W // CHUNK
NBUF = 5
LAG = 2
N_GROUPS = N_CHUNKS // NBUF


def _make_gather():
    mesh = plsc.VectorSubcoreMesh(core_axis_name="c", subcore_axis_name="s")

    @functools.partial(
        pl.kernel,
        mesh=mesh,
        out_type=jax.ShapeDtypeStruct((TOTAL, DIM), jnp.float32),
        scratch_types=[
            pltpu.VMEM((N_CHUNKS, CHUNK), jnp.int32),
            pltpu.VMEM((NBUF, CHUNK, DIM), jnp.float32),
            pltpu.SemaphoreType.DMA((NBUF,)),
            pltpu.SemaphoreType.DMA((NBUF,)),
        ],
    )
    def gather_kernel(table_hbm, ids_hbm, out_hbm, idx_all, rows, sem_g, sem_s):
        wid = lax.axis_index("s") * NC + lax.axis_index("c")
        base = wid * PER_W

        pltpu.sync_copy(ids_hbm.at[wid], idx_all)

        def gather(k, b):
            pltpu.async_copy(table_hbm.at[idx_all.at[k]], rows.at[b], sem_g.at[b])

        def wait_gather(k, b):
            pltpu.make_async_copy(
                table_hbm.at[idx_all.at[k]], rows.at[b], sem_g.at[b]).wait()

        def store(k, b):
            pltpu.async_copy(
                rows.at[b], out_hbm.at[pl.ds(base + k * CHUNK, CHUNK)], sem_s.at[b])

        def wait_store(k, b):
            pltpu.make_async_copy(
                rows.at[b], out_hbm.at[pl.ds(base + k * CHUNK, CHUNK)],
                sem_s.at[b]).wait()

        for k in range(LAG):
            gather(k, k)
        for k in range(LAG, NBUF):
            gather(k, k)
            wait_gather(k - LAG, k - LAG)
            store(k - LAG, k - LAG)

        def body(j, _):
            k0 = j * NBUF
            for b in range(NBUF):
                k = k0 + b
                wait_store(k - NBUF, b)
                gather(k, b)
                kl = k - LAG
                bl = (b - LAG) % NBUF
                wait_gather(kl, bl)
                store(kl, bl)
            return 0

        lax.fori_loop(1, N_GROUPS, body, 0)

        for k in range(N_CHUNKS - LAG, N_CHUNKS):
            wait_gather(k, k % NBUF)
            store(k, k % NBUF)
        for k in range(N_CHUNKS - NBUF, N_CHUNKS):
            wait_store(k, k % NBUF)

    return gather_kernel


_gather = _make_gather()


def kernel(input_ids, table):
    ids = input_ids.reshape(NW, N_CHUNKS, CHUNK).astype(jnp.int32)
    out_flat = _gather(table, ids)
    return out_flat.reshape(B, L, DIM)
